# baseline (speedup 1.0000x reference)
import jax
import jax.numpy as jnp
from jax.experimental import pallas as pl
from jax.experimental.pallas import tpu as pltpu

_VIEW_COLS = 2048
_GROUPS = _VIEW_COLS // 64
_BLOCK_ROWS = 64


def _rowdot_kernel(gu_ref, gi_ref, out_ref):
    p = gu_ref[:] * gi_ref[:]
    out_ref[:] = jnp.sum(p.reshape(p.shape[0], _GROUPS, 64), axis=2)


def kernel(gu, gi):
    n, k = gu.shape
    rows = n * k // _VIEW_COLS
    gu2 = gu.reshape(rows, _VIEW_COLS)
    gi2 = gi.reshape(rows, _VIEW_COLS)
    grid = (rows // _BLOCK_ROWS,)
    out = pl.pallas_call(
        _rowdot_kernel,
        grid=grid,
        in_specs=[
            pl.BlockSpec((_BLOCK_ROWS, _VIEW_COLS), lambda i: (i, 0)),
            pl.BlockSpec((_BLOCK_ROWS, _VIEW_COLS), lambda i: (i, 0)),
        ],
        out_specs=pl.BlockSpec((_BLOCK_ROWS, _GROUPS), lambda i: (i, 0)),
        out_shape=jax.ShapeDtypeStruct((rows, _GROUPS), jnp.float32),
    )(gu2, gi2)
    return out.reshape(n)


# speedup vs baseline: 1.6109x; 1.6109x over previous
import jax
import jax.numpy as jnp
from jax.experimental import pallas as pl

_SPLIT = 4
_SUB_ROWS = 512
_STEP_ROWS = _SPLIT * _SUB_ROWS


def _rowdot_kernel(*refs):
    out_ref = refs[-1]
    gu_refs = refs[:_SPLIT]
    gi_refs = refs[_SPLIT:2 * _SPLIT]
    for j in range(_SPLIT):
        p = gu_refs[j][:] * gi_refs[j][:]
        out_ref[pl.ds(j * _SUB_ROWS, _SUB_ROWS)] = jnp.sum(p, axis=1)


def _make_in_spec(j):
    return pl.BlockSpec((_SUB_ROWS, 64), lambda i, j=j: (_SPLIT * i + j, 0))


def kernel(gu, gi):
    n, k = gu.shape
    grid = (n // _STEP_ROWS,)
    out = pl.pallas_call(
        _rowdot_kernel,
        grid=grid,
        in_specs=[_make_in_spec(j) for j in range(_SPLIT)] * 2,
        out_specs=pl.BlockSpec((_STEP_ROWS,), lambda i: (i,)),
        out_shape=jax.ShapeDtypeStruct((n,), jnp.float32),
    )(*([gu] * _SPLIT + [gi] * _SPLIT))
    return out
